# Initial kernel scaffold; baseline (speedup 1.0000x reference)
import jax
import jax.numpy as jnp
from jax.experimental import pallas as pl


def kernel(x, src1, dst1, src2, dst2, W1_l, b1_l, W1_r, W2_l, b2_l, W2_r):
    raise NotImplementedError("write your pallas kernel here")


# speedup vs baseline: 5.2231x; 5.2231x over previous
import functools

import jax
import jax.numpy as jnp
from jax import lax
from jax.experimental import pallas as pl
from jax.experimental.pallas import tpu as pltpu
from jax.experimental.pallas import tpu_sc as plsc

N = 10000
N1 = 2000
N2 = 500
E1 = 320000
E2 = 64000
D = 128

NC = 2
NS = 16
NW = NC * NS
BLK = 128


def _ceil_to(a, m):
    return (a + m - 1) // m * m


def _make_sc_agg(nblk, AR):
    SR = AR // NS
    mesh = plsc.VectorSubcoreMesh(core_axis_name="c", subcore_axis_name="s")

    @functools.partial(
        pl.kernel,
        mesh=mesh,
        out_type=[
            jax.ShapeDtypeStruct((NC, AR, D), jnp.float32),
            jax.ShapeDtypeStruct((NC, AR, 16), jnp.float32),
        ],
        scratch_types=[
            pltpu.VMEM((nblk, BLK), jnp.int32),
            pltpu.VMEM((nblk, BLK), jnp.int32),
            pltpu.VMEM((BLK, D), jnp.float32),
            pltpu.VMEM((BLK, 16), jnp.float32),
            pltpu.VMEM_SHARED((AR, D), jnp.float32),
            pltpu.VMEM_SHARED((AR, 16), jnp.float32),
            pltpu.SemaphoreType.DMA,
        ],
    )
    def k(table, srcs, dsts, ones, zr, zc, acc_out, cnt_out,
          src_v, dst_v, rows_v, ones_v, acc_s, cnt_s, sem):
        cid = lax.axis_index("c")
        sid = lax.axis_index("s")
        wid = sid * NC + cid

        pltpu.sync_copy(zr.at[pl.ds(0, SR)], acc_s.at[pl.ds(sid * SR, SR)])
        pltpu.sync_copy(zc.at[pl.ds(0, SR)], cnt_s.at[pl.ds(sid * SR, SR)])

        pltpu.sync_copy(srcs.at[wid], src_v)
        pltpu.sync_copy(dsts.at[wid], dst_v)
        pltpu.sync_copy(ones, ones_v)
        plsc.subcore_barrier()

        def body(j, carry):
            pltpu.async_copy(table.at[src_v.at[j]], rows_v, sem).wait()
            pltpu.sync_copy(rows_v, acc_s.at[dst_v.at[j]], add=True)
            pltpu.sync_copy(ones_v, cnt_s.at[dst_v.at[j]], add=True)
            return carry

        lax.fori_loop(0, nblk, body, 0)

        plsc.subcore_barrier()

        @pl.when(sid == 0)
        def _():
            pltpu.sync_copy(acc_s, acc_out.at[cid])
            pltpu.sync_copy(cnt_s, cnt_out.at[cid])

    return k


def _tc_layer1(acc, cnt, x, wl, bl, wr):
    def body(acc_r, cnt_r, x_r, wl_r, bl_r, wr_r, o_r):
        s = acc_r[0] + acc_r[1]
        c = jnp.sum(cnt_r[0] + cnt_r[1], axis=-1, keepdims=True)
        mean = s / jnp.maximum(c, 1.0)
        h = (jnp.dot(mean, wl_r[...], preferred_element_type=jnp.float32)
             + bl_r[...]
             + jnp.dot(x_r[...], wr_r[...], preferred_element_type=jnp.float32))
        o_r[...] = jnp.maximum(h, 0.0)

    return pl.pallas_call(
        body,
        out_shape=jax.ShapeDtypeStruct((acc.shape[1], D), jnp.float32),
    )(acc, cnt, x, wl, bl, wr)


def _tc_layer2(acc, cnt, h, wl, bl, wr):
    def body(acc_r, cnt_r, h_r, wl_r, bl_r, wr_r, o_r):
        s = acc_r[0] + acc_r[1]
        c = jnp.sum(cnt_r[0] + cnt_r[1], axis=-1, keepdims=True)
        mean = s / jnp.maximum(c, 1.0)
        z = (jnp.dot(mean, wl_r[...], preferred_element_type=jnp.float32)
             + bl_r[...]
             + jnp.dot(h_r[...], wr_r[...], preferred_element_type=jnp.float32))
        m = jnp.max(z, axis=-1, keepdims=True)
        e = z - m
        lse = jnp.log(jnp.sum(jnp.exp(e), axis=-1, keepdims=True))
        o_r[...] = e - lse

    return pl.pallas_call(
        body,
        out_shape=jax.ShapeDtypeStruct((acc.shape[1], D), jnp.float32),
    )(acc, cnt, h, wl, bl, wr)


def _pad_edges(src, dst, pad_dst, ep):
    e = src.shape[0]
    tot = NW * ep
    src_p = jnp.concatenate(
        [src, jnp.zeros((tot - e,), jnp.int32)]).reshape(NW, ep // BLK, BLK)
    dst_p = jnp.concatenate(
        [dst, jnp.full((tot - e,), pad_dst, jnp.int32)]).reshape(NW, ep // BLK, BLK)
    return src_p, dst_p


def kernel(x, src1, dst1, src2, dst2, W1_l, b1_l, W1_r, W2_l, b2_l, W2_r):
    AR1, AR2 = 2048, 512
    ep1 = _ceil_to(E1 // NW, BLK)
    ep2 = _ceil_to(E2 // NW, BLK)

    srcs1, dsts1 = _pad_edges(src1, dst1, AR1 - 1, ep1)
    srcs2, dsts2 = _pad_edges(src2, dst2, AR2 - 1, ep2)

    ones = jnp.zeros((BLK, 16), jnp.float32).at[:, 0].set(1.0)
    zr = jnp.zeros((AR1 // NS, D), jnp.float32)
    zc = jnp.zeros((AR1 // NS, 16), jnp.float32)

    agg1 = _make_sc_agg(ep1 // BLK, AR1)
    acc1, cnt1 = agg1(x, srcs1, dsts1, ones, zr, zc)

    h = _tc_layer1(acc1, cnt1, x[:AR1], W1_l, b1_l.reshape(1, D), W1_r)

    agg2 = _make_sc_agg(ep2 // BLK, AR2)
    acc2, cnt2 = agg2(h, srcs2, dsts2, ones, zr[: AR2 // NS], zc[: AR2 // NS])

    out = _tc_layer2(acc2, cnt2, h[:AR2], W2_l, b2_l.reshape(1, D), W2_r)
    return out[:N2]
